# Initial kernel scaffold; baseline (speedup 1.0000x reference)
import jax
import jax.numpy as jnp
from jax.experimental import pallas as pl


def kernel(x, edge_index, W1, b1, W2, b2):
    raise NotImplementedError("write your pallas kernel here")


# speedup vs baseline: 12.1502x; 12.1502x over previous
import functools

import jax
import jax.numpy as jnp
from jax import lax
from jax.experimental import pallas as pl
from jax.experimental.pallas import tpu as pltpu
from jax.experimental.pallas import tpu_sc as plsc

NC = 2
NS = 16
L = 16


def _sc_mesh():
    return plsc.VectorSubcoreMesh(
        core_axis_name="c", subcore_axis_name="s", num_cores=NC, num_subcores=NS
    )


def _make_deg_kernel(E, n, NP, W):
    NW = NC * NS
    EPW = E // NW
    C = 80
    assert E % NW == 0 and EPW % C == 0 and W % L == 0
    RPT = NP // NS
    ZR = 32
    assert NP % NS == 0 and RPT % ZR == 0

    @functools.partial(
        pl.kernel,
        out_type=jax.ShapeDtypeStruct((NC * NP, W), jnp.float32),
        mesh=_sc_mesh(),
        scratch_types=[
            pltpu.VMEM((C,), jnp.int32),
            pltpu.VMEM((C, W), jnp.float32),
            pltpu.VMEM((ZR, W), jnp.float32),
            pltpu.VMEM_SHARED((NP, W), jnp.float32),
            pltpu.SemaphoreType.DMA,
        ],
    )
    def deg_kernel(dst_hbm, out_hbm, idx_v, ones_v, zb_v, acc_sh, sem):
        c = lax.axis_index("c")
        s = lax.axis_index("s")
        wid = c * NS + s
        one_row = jnp.where(lax.iota(jnp.int32, L) == 0, 1.0, 0.0)
        zero_row = jnp.zeros((L,), jnp.float32)

        @pl.loop(0, C)
        def _(i):
            ones_v[i, pl.ds(0, L)] = one_row

            @pl.loop(L, W, step=L)
            def _(j):
                ones_v[i, pl.ds(j, L)] = zero_row

        @pl.loop(0, ZR)
        def _(i):
            @pl.loop(0, W, step=L)
            def _(j):
                zb_v[i, pl.ds(j, L)] = zero_row

        @pl.loop(0, RPT, step=ZR)
        def _(r):
            pltpu.sync_copy(zb_v, acc_sh.at[pl.ds(s * RPT + r, ZR)])

        plsc.subcore_barrier()

        @pl.loop(0, EPW, step=C)
        def _(i):
            pltpu.sync_copy(dst_hbm.at[pl.ds(wid * EPW + i, C)], idx_v)
            pltpu.sync_copy(ones_v, acc_sh.at[idx_v], add=True)

        plsc.subcore_barrier()
        pltpu.sync_copy(
            acc_sh.at[pl.ds(s * RPT, RPT)],
            out_hbm.at[pl.ds(c * NP + s * RPT, RPT)],
        )

    return deg_kernel


def _make_mp_kernel(E, n, D, NP):
    NW = NC * NS
    EPW = E // NW
    C = 80
    assert E % NW == 0 and EPW % C == 0 and D % L == 0
    RPT = NP // NS
    ZR = 32
    assert NP % NS == 0 and RPT % ZR == 0

    @functools.partial(
        pl.kernel,
        out_type=jax.ShapeDtypeStruct((NC * NP, D), jnp.float32),
        mesh=_sc_mesh(),
        scratch_types=[
            pltpu.VMEM((C,), jnp.int32),
            pltpu.VMEM((C,), jnp.int32),
            pltpu.VMEM((C, D), jnp.float32),
            pltpu.VMEM((ZR, D), jnp.float32),
            pltpu.VMEM_SHARED((NP, D), jnp.float32),
            pltpu.SemaphoreType.DMA,
        ],
    )
    def mp_kernel(g_hbm, src_hbm, dst_hbm, out_hbm,
                  srci_v, dsti_v, rows_v, zb_v, acc_sh, sem):
        c = lax.axis_index("c")
        s = lax.axis_index("s")
        wid = c * NS + s
        zero_row = jnp.zeros((L,), jnp.float32)

        @pl.loop(0, ZR)
        def _(i):
            @pl.loop(0, D, step=L)
            def _(j):
                zb_v[i, pl.ds(j, L)] = zero_row

        @pl.loop(0, RPT, step=ZR)
        def _(r):
            pltpu.sync_copy(zb_v, acc_sh.at[pl.ds(s * RPT + r, ZR)])

        plsc.subcore_barrier()

        @pl.loop(0, EPW, step=C)
        def _(i):
            base = wid * EPW + i
            pltpu.sync_copy(src_hbm.at[pl.ds(base, C)], srci_v)
            pltpu.sync_copy(dst_hbm.at[pl.ds(base, C)], dsti_v)
            pltpu.async_copy(g_hbm.at[srci_v], rows_v, sem).wait()
            pltpu.sync_copy(rows_v, acc_sh.at[dsti_v], add=True)

        plsc.subcore_barrier()
        pltpu.sync_copy(
            acc_sh.at[pl.ds(s * RPT, RPT)],
            out_hbm.at[pl.ds(c * NP + s * RPT, RPT)],
        )

    return mp_kernel


def _mm_body(x_ref, w_ref, o_ref):
    o_ref[...] = jnp.dot(x_ref[...], w_ref[...],
                         preferred_element_type=jnp.float32)


def _scale_body(h_ref, dp_ref, g_ref, dinv_ref):
    p = dp_ref[...]
    deg = jnp.sum(p[0] + p[1], axis=1, keepdims=True) + 1.0
    dinv = lax.rsqrt(deg)
    dinv_ref[...] = dinv
    g_ref[...] = dinv * h_ref[...]


def _mid_body(a_ref, g_ref, dinv_ref, b_ref, w_ref, o_ref):
    dinv = dinv_ref[...]
    acc = a_ref[0] + a_ref[1] + g_ref[...]
    z = jnp.maximum(dinv * acc + b_ref[...], 0.0)
    o_ref[...] = dinv * jnp.dot(z, w_ref[...],
                                preferred_element_type=jnp.float32)


def _out_body(a_ref, g_ref, dinv_ref, b_ref, o_ref):
    o_ref[...] = dinv_ref[...] * (a_ref[0] + a_ref[1] + g_ref[...]) + b_ref[...]


def kernel(x, edge_index, W1, b1, W2, b2):
    n, K = x.shape
    E = edge_index.shape[1]
    RB = 1000
    assert n % RB == 0
    G = n // RB
    src = edge_index[0]
    dst = edge_index[1]
    NP = -(-n // 2048) * 2048

    deg_parts = _make_deg_kernel(E, n, NP, K)(dst)

    h1 = pl.pallas_call(
        _mm_body,
        grid=(G,),
        in_specs=[pl.BlockSpec((RB, K), lambda i: (i, 0)),
                  pl.BlockSpec((K, K), lambda i: (0, 0))],
        out_specs=pl.BlockSpec((RB, K), lambda i: (i, 0)),
        out_shape=jax.ShapeDtypeStruct((n, K), jnp.float32),
    )(x, W1)

    g1, dinv = pl.pallas_call(
        _scale_body,
        grid=(G,),
        in_specs=[pl.BlockSpec((RB, K), lambda i: (i, 0)),
                  pl.BlockSpec((NC, RB, K), lambda i: (0, i, 0))],
        out_specs=[pl.BlockSpec((RB, K), lambda i: (i, 0)),
                   pl.BlockSpec((RB, 1), lambda i: (i, 0))],
        out_shape=[jax.ShapeDtypeStruct((n, K), jnp.float32),
                   jax.ShapeDtypeStruct((n, 1), jnp.float32)],
    )(h1, deg_parts.reshape(NC, NP, K)[:, :n, :])

    mp = _make_mp_kernel(E, n, K, NP)
    acc1 = mp(g1, src, dst).reshape(NC, NP, K)[:, :n, :]

    g2 = pl.pallas_call(
        _mid_body,
        grid=(G,),
        in_specs=[pl.BlockSpec((NC, RB, K), lambda i: (0, i, 0)),
                  pl.BlockSpec((RB, K), lambda i: (i, 0)),
                  pl.BlockSpec((RB, 1), lambda i: (i, 0)),
                  pl.BlockSpec((1, K), lambda i: (0, 0)),
                  pl.BlockSpec((K, K), lambda i: (0, 0))],
        out_specs=pl.BlockSpec((RB, K), lambda i: (i, 0)),
        out_shape=jax.ShapeDtypeStruct((n, K), jnp.float32),
    )(acc1, g1, dinv, b1.reshape(1, K), W2)

    acc2 = mp(g2, src, dst).reshape(NC, NP, K)[:, :n, :]

    out = pl.pallas_call(
        _out_body,
        grid=(G,),
        in_specs=[pl.BlockSpec((NC, RB, K), lambda i: (0, i, 0)),
                  pl.BlockSpec((RB, K), lambda i: (i, 0)),
                  pl.BlockSpec((RB, 1), lambda i: (i, 0)),
                  pl.BlockSpec((1, K), lambda i: (0, 0))],
        out_specs=pl.BlockSpec((RB, K), lambda i: (i, 0)),
        out_shape=jax.ShapeDtypeStruct((n, K), jnp.float32),
    )(acc2, g2, dinv, b2.reshape(1, K))

    return out
